# baseline (speedup 1.0000x reference)
import functools

import jax
import jax.numpy as jnp
from jax import lax
from jax.experimental import pallas as pl
from jax.experimental.pallas import tpu as pltpu
from jax.experimental.pallas import tpu_sc as plsc

_TOPK = 50
_MM_DTYPE = jnp.bfloat16
_C = 64


def _attn_body(x_ref, wk_ref, wv_ref, at_ref):
    C = _C
    X2b = x_ref[0].astype(_MM_DTYPE)
    dn = (((1,), (1,)), ((), ()))
    k2 = lax.dot_general(x_ref[0].astype(_MM_DTYPE), wk_ref[...].astype(_MM_DTYPE),
                         dn, preferred_element_type=jnp.float32)
    v2 = lax.dot_general(X2b, wv_ref[...].astype(_MM_DTYPE),
                         dn, preferred_element_type=jnp.float32)
    afT = lax.dot_general(
        k2.astype(_MM_DTYPE), v2.astype(_MM_DTYPE),
        (((0,), (0,)), ((), ())), preferred_element_type=jnp.float32,
    )
    at_ref[0] = (afT[0:C, 0:C] + afT[C:2 * C, C:2 * C]) * (C ** -0.5)


def _mask_sc_body(at_hbm, pt_hbm, abuf, ebuf):
    C = _C
    wid = lax.axis_index("s") * 2 + lax.axis_index("c")
    b = wid // 4
    off = (wid % 4) * 16
    pltpu.sync_copy(at_hbm.at[b], abuf)

    one = jnp.full((16,), 1.0, jnp.float32)
    zero = jnp.zeros((16,), jnp.float32)
    ninf = jnp.full((16,), -jnp.inf, jnp.float32)
    topk = jnp.full((16,), float(_TOPK), jnp.float32)

    JB = 8

    def rank_block(blk, m):
        j0 = blk * JB
        ajs = [abuf[j0 + t, pl.ds(off, 16)] for t in range(JB)]
        jvs = [zero + (j0 + t).astype(jnp.float32) for t in range(JB)]
        accs = [zero] * JB
        for jp in range(C):
            ajp = abuf[jp, pl.ds(off, 16)]
            jpv = jnp.full((16,), float(jp), jnp.float32)
            for t in range(JB):
                lower_f = jnp.where(jpv < jvs[t], one, zero)
                gt_f = jnp.where(ajp > ajs[t], one, zero)
                eq_f = jnp.where(ajp == ajs[t], lower_f, zero)
                accs[t] = accs[t] + gt_f + eq_f
        for t in range(JB):
            masked = jnp.where(accs[t] < topk, ajs[t], ninf)
            ebuf[j0 + t, pl.ds(0, 16)] = masked
            m = jnp.maximum(m, masked)
        return m

    m = lax.fori_loop(0, C // JB, rank_block, ninf)

    def expsum_row(j, s):
        e = jnp.exp(ebuf[j, pl.ds(0, 16)] - m)
        ebuf[j, pl.ds(0, 16)] = e
        return s + e

    s = lax.fori_loop(0, C, expsum_row, zero)
    r = one / s

    def norm_row(j, carry):
        ebuf[j, pl.ds(0, 16)] = ebuf[j, pl.ds(0, 16)] * r
        return carry

    lax.fori_loop(0, C, norm_row, 0)

    pltpu.sync_copy(ebuf, pt_hbm.at[wid])


def _out_body(x_ref, pt_ref, wq_ref, v0_ref, v1_ref, bp_ref, o_ref):
    C = _C
    X2 = x_ref[0]
    X2b = X2.astype(_MM_DTYPE)

    G = lax.dot_general(
        pt_ref[0].astype(_MM_DTYPE), wq_ref[...].astype(_MM_DTYPE),
        (((0,), (0,)), ((), ())), preferred_element_type=jnp.float32,
    )

    X2r = X2b.reshape(128, C, 128)
    V0 = v0_ref[...].astype(_MM_DTYPE)
    V1 = v1_ref[...].astype(_MM_DTYPE)
    dnt = (((1,), (0,)), ((), ()))
    T0 = lax.dot_general(X2r, V0, dnt, preferred_element_type=jnp.float32)
    T1 = lax.dot_general(X2r, V1, dnt, preferred_element_type=jnp.float32)
    QPv = T0[:, 0:C, :] + T1[:, C:2 * C, :]

    out = lax.dot_general(
        G.astype(_MM_DTYPE), QPv.astype(_MM_DTYPE),
        (((1,), (1,)), ((), ())), preferred_element_type=jnp.float32,
    )
    o_ref[0] = out + bp_ref[...][0][None, None, :] + X2.reshape(C, 128, 128)


def kernel(x, W_qkv, W_proj, b_proj):
    B, C, H, W = x.shape
    z = jnp.zeros((C, C), W_qkv.dtype)

    def bd(w):
        return jnp.block([[w, z], [z, w]])

    Wq = W_qkv[0:C]
    Wk = bd(W_qkv[C:2 * C])
    Wv = bd(W_qkv[2 * C:3 * C])
    Wpbd = bd(W_proj)
    V0 = Wpbd[:, 0::2].T
    V1 = Wpbd[:, 1::2].T
    bp2 = jnp.concatenate([b_proj, b_proj]).reshape(1, 2 * C)

    x2 = x.reshape(B, H * W * C // 128, 128)
    NP = H * W * C // 128

    HB = B // 2
    mesh = plsc.VectorSubcoreMesh(core_axis_name="c", subcore_axis_name="s")
    mask_sc = functools.partial(
        pl.kernel,
        mesh=mesh,
        out_type=jax.ShapeDtypeStruct((4 * HB, C, 16), jnp.float32),
        scratch_types=[
            pltpu.VMEM((C, C), jnp.float32),
            pltpu.VMEM((C, 16), jnp.float32),
        ],
    )(_mask_sc_body)

    def attn_half(x2h):
        return pl.pallas_call(
            _attn_body,
            grid=(HB,),
            in_specs=[
                pl.BlockSpec((1, NP, 128), lambda b: (b, 0, 0)),
                pl.BlockSpec((2 * C, 2 * C), lambda b: (0, 0)),
                pl.BlockSpec((2 * C, 2 * C), lambda b: (0, 0)),
            ],
            out_specs=pl.BlockSpec((1, C, C), lambda b: (b, 0, 0)),
            out_shape=jax.ShapeDtypeStruct((HB, C, C), jnp.float32),
        )(x2h, Wk, Wv)

    pt4 = jnp.concatenate(
        [mask_sc(attn_half(x2[:HB])), mask_sc(attn_half(x2[HB:]))], axis=0
    )

    pt = (pt4.reshape(B, 4, C, 16)
          .transpose(0, 2, 1, 3)
          .reshape(B, C, C))

    out = pl.pallas_call(
        _out_body,
        grid=(B,),
        in_specs=[
            pl.BlockSpec((1, NP, 128), lambda b: (b, 0, 0)),
            pl.BlockSpec((1, C, C), lambda b: (b, 0, 0)),
            pl.BlockSpec((C, C), lambda b: (0, 0)),
            pl.BlockSpec((C, 2 * C), lambda b: (0, 0)),
            pl.BlockSpec((C, 2 * C), lambda b: (0, 0)),
            pl.BlockSpec((1, 2 * C), lambda b: (0, 0)),
        ],
        out_specs=pl.BlockSpec((1, C, H, W), lambda b: (b, 0, 0, 0)),
        out_shape=jax.ShapeDtypeStruct((B, C, H, W), jnp.float32),
    )(x2, pt, Wq, V0, V1, bp2)
    return out


# speedup vs baseline: 1.2375x; 1.2375x over previous
import functools

import jax
import jax.numpy as jnp
from jax import lax
from jax.experimental import pallas as pl
from jax.experimental.pallas import tpu as pltpu
from jax.experimental.pallas import tpu_sc as plsc

_TOPK = 50
_MM_DTYPE = jnp.bfloat16
_C = 64


def _attn_body(x_ref, wk_ref, wv_ref, at_ref):
    C = _C
    X2b = x_ref[0].astype(_MM_DTYPE)
    dn = (((1,), (1,)), ((), ()))
    k2 = lax.dot_general(x_ref[0].astype(_MM_DTYPE), wk_ref[...].astype(_MM_DTYPE),
                         dn, preferred_element_type=jnp.float32)
    v2 = lax.dot_general(X2b, wv_ref[...].astype(_MM_DTYPE),
                         dn, preferred_element_type=jnp.float32)
    afT = lax.dot_general(
        k2.astype(_MM_DTYPE), v2.astype(_MM_DTYPE),
        (((0,), (0,)), ((), ())), preferred_element_type=jnp.float32,
    )
    at_ref[0] = (afT[0:C, 0:C] + afT[C:2 * C, C:2 * C]) * (C ** -0.5)


def _mask_sc_body(at_hbm, pt_hbm, abuf, ebuf):
    C = _C
    wid = lax.axis_index("s") * 2 + lax.axis_index("c")
    b = wid // 2
    half = wid % 2
    pltpu.sync_copy(at_hbm.at[b], abuf)

    one = jnp.full((16,), 1.0, jnp.float32)
    zero = jnp.zeros((16,), jnp.float32)
    ninf = jnp.full((16,), -jnp.inf, jnp.float32)
    topk = jnp.full((16,), float(_TOPK), jnp.float32)

    for c in range(2):
        off = half * 32 + c * 16

        JB = 8

        def rank_block(blk, m):
            j0 = blk * JB
            ajs = [abuf[j0 + t, pl.ds(off, 16)] for t in range(JB)]
            jvs = [zero + (j0 + t).astype(jnp.float32) for t in range(JB)]
            accs = [zero] * JB
            for jp in range(C):
                ajp = abuf[jp, pl.ds(off, 16)]
                jpv = jnp.full((16,), float(jp), jnp.float32)
                for t in range(JB):
                    lower_f = jnp.where(jpv < jvs[t], one, zero)
                    gt_f = jnp.where(ajp > ajs[t], one, zero)
                    eq_f = jnp.where(ajp == ajs[t], lower_f, zero)
                    accs[t] = accs[t] + gt_f + eq_f
            for t in range(JB):
                masked = jnp.where(accs[t] < topk, ajs[t], ninf)
                ebuf[j0 + t, pl.ds(c * 16, 16)] = masked
                m = jnp.maximum(m, masked)
            return m

        m = lax.fori_loop(0, C // JB, rank_block, ninf)

        def expsum_row(j, s):
            e = jnp.exp(ebuf[j, pl.ds(c * 16, 16)] - m)
            ebuf[j, pl.ds(c * 16, 16)] = e
            return s + e

        s = lax.fori_loop(0, C, expsum_row, zero)
        r = one / s

        def norm_row(j, carry):
            ebuf[j, pl.ds(c * 16, 16)] = ebuf[j, pl.ds(c * 16, 16)] * r
            return carry

        lax.fori_loop(0, C, norm_row, 0)

    pltpu.sync_copy(ebuf, pt_hbm.at[wid])


def _out_body(x_ref, pt_ref, wq_ref, v0_ref, v1_ref, bp_ref, o_ref):
    C = _C
    X2 = x_ref[0]
    X2b = X2.astype(_MM_DTYPE)

    G = lax.dot_general(
        pt_ref[0].astype(_MM_DTYPE), wq_ref[...].astype(_MM_DTYPE),
        (((0,), (0,)), ((), ())), preferred_element_type=jnp.float32,
    )

    X2r = X2b.reshape(128, C, 128)
    V0 = v0_ref[...].astype(_MM_DTYPE)
    V1 = v1_ref[...].astype(_MM_DTYPE)
    dnt = (((1,), (0,)), ((), ()))
    T0 = lax.dot_general(X2r, V0, dnt, preferred_element_type=jnp.float32)
    T1 = lax.dot_general(X2r, V1, dnt, preferred_element_type=jnp.float32)
    QPv = T0[:, 0:C, :] + T1[:, C:2 * C, :]

    out = lax.dot_general(
        G.astype(_MM_DTYPE), QPv.astype(_MM_DTYPE),
        (((1,), (1,)), ((), ())), preferred_element_type=jnp.float32,
    )
    o_ref[0] = out + bp_ref[...][0][None, None, :] + X2.reshape(C, 128, 128)


def kernel(x, W_qkv, W_proj, b_proj):
    B, C, H, W = x.shape
    z = jnp.zeros((C, C), W_qkv.dtype)

    def bd(w):
        return jnp.block([[w, z], [z, w]])

    Wq = W_qkv[0:C]
    Wk = bd(W_qkv[C:2 * C])
    Wv = bd(W_qkv[2 * C:3 * C])
    Wpbd = bd(W_proj)
    V0 = Wpbd[:, 0::2].T
    V1 = Wpbd[:, 1::2].T
    bp2 = jnp.concatenate([b_proj, b_proj]).reshape(1, 2 * C)

    x2 = x.reshape(B, H * W * C // 128, 128)
    NP = H * W * C // 128

    at = pl.pallas_call(
        _attn_body,
        grid=(B,),
        in_specs=[
            pl.BlockSpec((1, NP, 128), lambda b: (b, 0, 0)),
            pl.BlockSpec((2 * C, 2 * C), lambda b: (0, 0)),
            pl.BlockSpec((2 * C, 2 * C), lambda b: (0, 0)),
        ],
        out_specs=pl.BlockSpec((1, C, C), lambda b: (b, 0, 0)),
        out_shape=jax.ShapeDtypeStruct((B, C, C), jnp.float32),
    )(x2, Wk, Wv)

    mesh = plsc.VectorSubcoreMesh(core_axis_name="c", subcore_axis_name="s")
    mask_sc = functools.partial(
        pl.kernel,
        mesh=mesh,
        out_type=jax.ShapeDtypeStruct((2 * B, C, 32), jnp.float32),
        scratch_types=[
            pltpu.VMEM((C, C), jnp.float32),
            pltpu.VMEM((C, 32), jnp.float32),
        ],
    )(_mask_sc_body)
    pt4 = mask_sc(at)

    pt = (pt4.reshape(B, 2, C, 32)
          .transpose(0, 2, 1, 3)
          .reshape(B, C, C))

    out = pl.pallas_call(
        _out_body,
        grid=(B,),
        in_specs=[
            pl.BlockSpec((1, NP, 128), lambda b: (b, 0, 0)),
            pl.BlockSpec((1, C, C), lambda b: (b, 0, 0)),
            pl.BlockSpec((C, C), lambda b: (0, 0)),
            pl.BlockSpec((C, 2 * C), lambda b: (0, 0)),
            pl.BlockSpec((C, 2 * C), lambda b: (0, 0)),
            pl.BlockSpec((1, 2 * C), lambda b: (0, 0)),
        ],
        out_specs=pl.BlockSpec((1, C, H, W), lambda b: (b, 0, 0, 0)),
        out_shape=jax.ShapeDtypeStruct((B, C, H, W), jnp.float32),
    )(x2, pt, Wq, V0, V1, bp2)
    return out


# speedup vs baseline: 1.2695x; 1.0258x over previous
import functools

import jax
import jax.numpy as jnp
from jax import lax
from jax.experimental import pallas as pl
from jax.experimental.pallas import tpu as pltpu
from jax.experimental.pallas import tpu_sc as plsc

_TOPK = 50
_MM_DTYPE = jnp.bfloat16
_C = 64


def _attn_body(x_ref, wk_ref, wv_ref, at_ref):
    C = _C
    X2b = x_ref[0].astype(_MM_DTYPE)
    dn = (((1,), (1,)), ((), ()))
    k2 = lax.dot_general(x_ref[0].astype(_MM_DTYPE), wk_ref[...].astype(_MM_DTYPE),
                         dn, preferred_element_type=jnp.float32)
    v2 = lax.dot_general(X2b, wv_ref[...].astype(_MM_DTYPE),
                         dn, preferred_element_type=jnp.float32)
    afT = lax.dot_general(
        k2.astype(_MM_DTYPE), v2.astype(_MM_DTYPE),
        (((0,), (0,)), ((), ())), preferred_element_type=jnp.float32,
    )
    at_ref[0] = (afT[0:C, 0:C] + afT[C:2 * C, C:2 * C]) * (C ** -0.5)


def _mask_sc_body(at_hbm, pt_hbm, abuf, ebuf):
    C = _C
    wid = lax.axis_index("s") * 2 + lax.axis_index("c")
    b = wid // 2
    half = wid % 2
    pltpu.sync_copy(at_hbm.at[b], abuf)

    one = jnp.full((16,), 1.0, jnp.float32)
    zero = jnp.zeros((16,), jnp.float32)
    ninf = jnp.full((16,), -jnp.inf, jnp.float32)
    topk = jnp.full((16,), float(_TOPK), jnp.float32)

    for c in range(2):
        off = half * 32 + c * 16

        JB = 8

        def rank_block(blk, m):
            j0 = blk * JB
            ajs = [abuf[j0 + t, pl.ds(off, 16)] for t in range(JB)]
            jvs = [zero + (j0 + t).astype(jnp.float32) for t in range(JB)]
            accs = [zero] * JB
            for jp in range(C):
                ajp = abuf[jp, pl.ds(off, 16)]
                jpv = jnp.full((16,), float(jp), jnp.float32)
                for t in range(JB):
                    lower_f = jnp.where(jpv < jvs[t], one, zero)
                    gt_f = jnp.where(ajp > ajs[t], one, zero)
                    eq_f = jnp.where(ajp == ajs[t], lower_f, zero)
                    accs[t] = accs[t] + gt_f + eq_f
            for t in range(JB):
                masked = jnp.where(accs[t] < topk, ajs[t], ninf)
                ebuf[j0 + t, pl.ds(c * 16, 16)] = masked
                m = jnp.maximum(m, masked)
            return m

        m = lax.fori_loop(0, C // JB, rank_block, ninf)

        def expsum_row(j, s):
            e = jnp.exp(ebuf[j, pl.ds(c * 16, 16)] - m)
            ebuf[j, pl.ds(c * 16, 16)] = e
            return s + e

        s = lax.fori_loop(0, C, expsum_row, zero)
        r = one / s

        def norm_row(j, carry):
            ebuf[j, pl.ds(c * 16, 16)] = ebuf[j, pl.ds(c * 16, 16)] * r
            return carry

        lax.fori_loop(0, C, norm_row, 0)

    pltpu.sync_copy(ebuf, pt_hbm.at[wid])


def _out_body(x_ref, pt_ref, wq_ref, v0_ref, v1_ref, bp_ref, o_ref):
    C = _C
    X2 = x_ref[0]
    X2b = X2.astype(_MM_DTYPE)

    G = lax.dot_general(
        pt_ref[0].astype(_MM_DTYPE), wq_ref[...].astype(_MM_DTYPE),
        (((0,), (0,)), ((), ())), preferred_element_type=jnp.float32,
    )

    X2r = X2b.reshape(128, C, 128)
    V0 = v0_ref[...].astype(_MM_DTYPE)
    V1 = v1_ref[...].astype(_MM_DTYPE)
    Vc = jnp.concatenate([V0, V1], axis=1)
    dnt = (((1,), (0,)), ((), ()))
    Tc = lax.dot_general(X2r, Vc, dnt, preferred_element_type=jnp.float32)
    QPv = Tc[:, 0:C, 0:128] + Tc[:, C:2 * C, 128:256]

    out = lax.dot_general(
        G.astype(_MM_DTYPE), QPv.astype(_MM_DTYPE),
        (((1,), (1,)), ((), ())), preferred_element_type=jnp.float32,
    )
    o_ref[0] = out + bp_ref[...][0][None, None, :] + X2.reshape(C, 128, 128)


def kernel(x, W_qkv, W_proj, b_proj):
    B, C, H, W = x.shape
    z = jnp.zeros((C, C), W_qkv.dtype)

    def bd(w):
        return jnp.block([[w, z], [z, w]])

    Wq = W_qkv[0:C]
    Wk = bd(W_qkv[C:2 * C])
    Wv = bd(W_qkv[2 * C:3 * C])
    Wpbd = bd(W_proj)
    V0 = Wpbd[:, 0::2].T
    V1 = Wpbd[:, 1::2].T
    bp2 = jnp.concatenate([b_proj, b_proj]).reshape(1, 2 * C)

    x2 = x.reshape(B, H * W * C // 128, 128)
    NP = H * W * C // 128

    at = pl.pallas_call(
        _attn_body,
        grid=(B,),
        in_specs=[
            pl.BlockSpec((1, NP, 128), lambda b: (b, 0, 0)),
            pl.BlockSpec((2 * C, 2 * C), lambda b: (0, 0)),
            pl.BlockSpec((2 * C, 2 * C), lambda b: (0, 0)),
        ],
        out_specs=pl.BlockSpec((1, C, C), lambda b: (b, 0, 0)),
        out_shape=jax.ShapeDtypeStruct((B, C, C), jnp.float32),
    )(x2, Wk, Wv)

    mesh = plsc.VectorSubcoreMesh(core_axis_name="c", subcore_axis_name="s")
    mask_sc = functools.partial(
        pl.kernel,
        mesh=mesh,
        out_type=jax.ShapeDtypeStruct((2 * B, C, 32), jnp.float32),
        scratch_types=[
            pltpu.VMEM((C, C), jnp.float32),
            pltpu.VMEM((C, 32), jnp.float32),
        ],
    )(_mask_sc_body)
    pt4 = mask_sc(at)

    pt = (pt4.reshape(B, 2, C, 32)
          .transpose(0, 2, 1, 3)
          .reshape(B, C, C))

    out = pl.pallas_call(
        _out_body,
        grid=(B,),
        in_specs=[
            pl.BlockSpec((1, NP, 128), lambda b: (b, 0, 0)),
            pl.BlockSpec((1, C, C), lambda b: (b, 0, 0)),
            pl.BlockSpec((C, C), lambda b: (0, 0)),
            pl.BlockSpec((C, 2 * C), lambda b: (0, 0)),
            pl.BlockSpec((C, 2 * C), lambda b: (0, 0)),
            pl.BlockSpec((1, 2 * C), lambda b: (0, 0)),
        ],
        out_specs=pl.BlockSpec((1, C, H, W), lambda b: (b, 0, 0, 0)),
        out_shape=jax.ShapeDtypeStruct((B, C, H, W), jnp.float32),
    )(x2, pt, Wq, V0, V1, bp2)
    return out
